# baseline (speedup 1.0000x reference)
import functools

import jax
import jax.numpy as jnp
from jax import lax
from jax.experimental import pallas as pl
from jax.experimental.pallas import tpu as pltpu
from jax.experimental.pallas import tpu_sc as plsc

_N = 4096
_M = 16384
_DIN = 256
_DSKIP = 128
_DOUT = 256
_K = 3


_TM = 512


def _knn_body(ps_ref, posT_ref, idx_ref, w_ref):
    ps = ps_ref[...]
    posT = posT_ref[...]
    qq = jnp.sum(ps * ps, axis=1, keepdims=True)
    kk = jnp.sum(posT * posT, axis=0, keepdims=True)
    cross = jnp.dot(ps, posT, preferred_element_type=jnp.float32)
    d2 = qq + kk - 2.0 * cross

    iota = lax.broadcasted_iota(jnp.int32, d2.shape, 1)
    big = jnp.float32(3.4e38)
    vals, idxs = [], []
    cur = d2
    for r in range(_K):
        m = jnp.min(cur, axis=1, keepdims=True)
        ival = jnp.min(jnp.where(cur <= m, iota, _N), axis=1, keepdims=True)
        vals.append(m)
        idxs.append(ival)
        if r + 1 < _K:
            cur = jnp.where(iota == ival, big, cur)

    wk = [1.0 / jnp.maximum(jnp.maximum(v, 0.0), 1e-16) for v in vals]
    wsum = wk[0] + wk[1] + wk[2]
    idx_ref[...] = jnp.concatenate(idxs, axis=1)
    w_ref[...] = jnp.concatenate([w_ / wsum for w_ in wk], axis=1)


def _knn(pos_skip, posT):
    m = pos_skip.shape[0]
    return pl.pallas_call(
        _knn_body,
        grid=(m // _TM,),
        in_specs=[
            pl.BlockSpec((_TM, 3), lambda i: (i, 0)),
            pl.BlockSpec((3, _N), lambda i: (0, 0)),
        ],
        out_specs=[
            pl.BlockSpec((_TM, _K), lambda i: (i, 0)),
            pl.BlockSpec((_TM, _K), lambda i: (i, 0)),
        ],
        out_shape=[
            jax.ShapeDtypeStruct((m, _K), jnp.int32),
            jax.ShapeDtypeStruct((m, _K), jnp.float32),
        ],
    )(pos_skip, posT)


_NW = 32
_CH = 32
_NBUF = 4


def _make_gather_body(m_slab):
    qw = m_slab // _NW
    nch = qw // _CH
    nbuf = min(_NBUF, nch)

    def body(x_hbm, idx_hbm, g_hbm, idx_v, rows_v, gsems, osems):
        wid = lax.axis_index("s") * 2 + lax.axis_index("c")
        r0 = wid * qw * 3

        def out_copy(c, b):
            return pltpu.make_async_copy(
                rows_v.at[b], g_hbm.at[pl.ds(r0 + c * _CH * 3, _CH * 3)],
                osems[b])

        def start_chunk(c, b):
            pltpu.sync_copy(idx_hbm.at[pl.ds(r0 + c * _CH * 3, _CH * 3)],
                            idx_v.at[b])
            pltpu.make_async_copy(x_hbm.at[idx_v.at[b]], rows_v.at[b],
                                  gsems[b]).start()

        for b in range(nbuf):
            start_chunk(b, b)
        for c in range(nch):
            b = c % nbuf
            pltpu.make_async_copy(x_hbm.at[idx_v.at[b]], rows_v.at[b],
                                  gsems[b]).wait()
            out_copy(c, b).start()
            if c + nbuf < nch:
                out_copy(c, b).wait()
                start_chunk(c + nbuf, b)
        for c in range(max(nch - nbuf, 0), nch):
            out_copy(c, c % nbuf).wait()

    return body


@functools.lru_cache(maxsize=2)
def _make_gather(m_slab):
    @functools.partial(
        pl.kernel,
        mesh=plsc.VectorSubcoreMesh(core_axis_name="c", subcore_axis_name="s"),
        out_type=jax.ShapeDtypeStruct((m_slab * _K, _DIN), jnp.float32),
        scratch_types=[
            pltpu.VMEM((_NBUF, _CH * 3), jnp.int32),
            pltpu.VMEM((_NBUF, _CH * 3, _DIN), jnp.float32),
            pltpu.SemaphoreType.DMA,
            pltpu.SemaphoreType.DMA,
            pltpu.SemaphoreType.DMA,
            pltpu.SemaphoreType.DMA,
            pltpu.SemaphoreType.DMA,
            pltpu.SemaphoreType.DMA,
            pltpu.SemaphoreType.DMA,
            pltpu.SemaphoreType.DMA,
        ],
    )
    def _gather(x_hbm, idx_hbm, g_hbm, idx_v, rows_v,
                g0, g1, g2, g3, o0, o1, o2, o3):
        _make_gather_body(m_slab)(x_hbm, idx_hbm, g_hbm, idx_v, rows_v,
                                  (g0, g1, g2, g3), (o0, o1, o2, o3))

    return _gather


_TMC = 1024


def _mlp_body(g_ref, w_ref, xs_ref, w1_ref, w2_ref, b_ref, o_ref):
    w = w_ref[...]
    xi = w[:, 0:1] * g_ref[:, 0, :]
    xi = xi + w[:, 1:2] * g_ref[:, 1, :]
    xi = xi + w[:, 2:3] * g_ref[:, 2, :]
    h = jnp.dot(xi, w1_ref[...], preferred_element_type=jnp.float32)
    h = h + jnp.dot(xs_ref[...], w2_ref[...], preferred_element_type=jnp.float32)
    o_ref[...] = jnp.maximum(h + b_ref[...], 0.0)


def _mlp(g, w, x_skip, W1, W2, b2d):
    m = w.shape[0]
    return pl.pallas_call(
        _mlp_body,
        grid=(m // _TMC,),
        in_specs=[
            pl.BlockSpec((_TMC, _K, _DIN), lambda i: (i, 0, 0)),
            pl.BlockSpec((_TMC, _K), lambda i: (i, 0)),
            pl.BlockSpec((_TMC, _DSKIP), lambda i: (i, 0)),
            pl.BlockSpec((_DIN, _DOUT), lambda i: (0, 0)),
            pl.BlockSpec((_DSKIP, _DOUT), lambda i: (0, 0)),
            pl.BlockSpec((1, _DOUT), lambda i: (0, 0)),
        ],
        out_specs=pl.BlockSpec((_TMC, _DOUT), lambda i: (i, 0)),
        out_shape=jax.ShapeDtypeStruct((m, _DOUT), jnp.float32),
    )(g.reshape(m, _K, _DIN), w, x_skip, W1, W2, b2d)


_NSLAB = 4


def kernel(x, pos, batch, x_skip, pos_skip, batch_skip, W, b):
    posT = pos.T
    W1, W2, b2d = W[:_DIN], W[_DIN:], b.reshape(1, _DOUT)
    ms = _M // _NSLAB

    gs, ws = [], []
    nxt = _knn(pos_skip[:ms], posT)
    for s in range(_NSLAB):
        idx, w = nxt
        gs.append(_make_gather(ms)(x, idx.reshape(-1)))
        ws.append(w)
        if s + 1 < _NSLAB:
            nxt = _knn(pos_skip[(s + 1) * ms:(s + 2) * ms], posT)
    outs = [_mlp(gs[s], ws[s], x_skip[s * ms:(s + 1) * ms], W1, W2, b2d)
            for s in range(_NSLAB)]
    out = jnp.concatenate(outs, axis=0)
    return (out, pos_skip, batch_skip)


# speedup vs baseline: 1.3169x; 1.3169x over previous
import functools

import jax
import jax.numpy as jnp
from jax import lax
from jax.experimental import pallas as pl
from jax.experimental.pallas import tpu as pltpu
from jax.experimental.pallas import tpu_sc as plsc

_N = 4096
_M = 16384
_DIN = 256
_DSKIP = 128
_DOUT = 256
_K = 3


_TM = 512


def _knn_body(ps_ref, posT_ref, idx_ref, w0_ref, w1_ref, w2_ref):
    ps = ps_ref[...]
    posT = posT_ref[...]
    qq = jnp.sum(ps * ps, axis=1, keepdims=True)
    kk = jnp.sum(posT * posT, axis=0, keepdims=True)
    cross = jnp.dot(ps, posT, preferred_element_type=jnp.float32)
    d2 = qq + kk - 2.0 * cross

    iota = lax.broadcasted_iota(jnp.int32, d2.shape, 1)
    big = jnp.float32(3.4e38)
    vals, idxs = [], []
    cur = d2
    for r in range(_K):
        m = jnp.min(cur, axis=1, keepdims=True)
        ival = jnp.min(jnp.where(cur <= m, iota, _N), axis=1, keepdims=True)
        vals.append(m)
        idxs.append(ival)
        if r + 1 < _K:
            cur = jnp.where(iota == ival, big, cur)

    wk = [1.0 / jnp.maximum(jnp.maximum(v, 0.0), 1e-16) for v in vals]
    wsum = wk[0] + wk[1] + wk[2]
    idx_ref[...] = jnp.concatenate(idxs, axis=1)
    w0_ref[...] = jnp.broadcast_to(wk[0] / wsum, (wk[0].shape[0], 16))
    w1_ref[...] = jnp.broadcast_to(wk[1] / wsum, (wk[1].shape[0], 16))
    w2_ref[...] = jnp.broadcast_to(wk[2] / wsum, (wk[2].shape[0], 16))


def _knn(pos_skip, posT):
    m = pos_skip.shape[0]
    wspec = pl.BlockSpec((_TM, 16), lambda i: (i, 0))
    wshape = jax.ShapeDtypeStruct((m, 16), jnp.float32)
    return pl.pallas_call(
        _knn_body,
        grid=(m // _TM,),
        in_specs=[
            pl.BlockSpec((_TM, 3), lambda i: (i, 0)),
            pl.BlockSpec((3, _N), lambda i: (0, 0)),
        ],
        out_specs=[
            pl.BlockSpec((_TM, _K), lambda i: (i, 0)),
            wspec, wspec, wspec,
        ],
        out_shape=[
            jax.ShapeDtypeStruct((m, _K), jnp.int32),
            wshape, wshape, wshape,
        ],
    )(pos_skip, posT)


_NW = 32
_CH = 32
_NBUF = 2


def _make_gather_body(m_slab):
    qw = m_slab // _NW
    nch = qw // _CH
    nbuf = min(_NBUF, nch)

    def body(x_hbm, idx_hbm, w0_hbm, w1_hbm, w2_hbm, xi_hbm,
             idx_v, w_v, rows_v, out_v, gsems, osems):
        wid = lax.axis_index("s") * 2 + lax.axis_index("c")
        q0 = wid * qw

        def out_copy(c, b):
            return pltpu.make_async_copy(
                out_v.at[b], xi_hbm.at[pl.ds(q0 + c * _CH, _CH)], osems[b])

        def start_chunk(c, b):
            qb = q0 + c * _CH
            pltpu.sync_copy(idx_hbm.at[pl.ds(qb * 3, _CH * 3)], idx_v.at[b])
            pltpu.sync_copy(w0_hbm.at[pl.ds(qb, _CH)], w_v.at[b, 0])
            pltpu.sync_copy(w1_hbm.at[pl.ds(qb, _CH)], w_v.at[b, 1])
            pltpu.sync_copy(w2_hbm.at[pl.ds(qb, _CH)], w_v.at[b, 2])
            pltpu.make_async_copy(x_hbm.at[idx_v.at[b]], rows_v.at[b],
                                  gsems[b]).start()

        def compute_chunk(c, b):
            pltpu.make_async_copy(x_hbm.at[idx_v.at[b]], rows_v.at[b],
                                  gsems[b]).wait()

            def q_body(q, carry):
                w0 = w_v[b, 0, q]
                w1 = w_v[b, 1, q]
                w2 = w_v[b, 2, q]
                for d in range(_DIN // 16):
                    sl = pl.ds(16 * d, 16)
                    out_v[b, q, sl] = (w0 * rows_v[b, 3 * q, sl]
                                       + w1 * rows_v[b, 3 * q + 1, sl]
                                       + w2 * rows_v[b, 3 * q + 2, sl])
                return carry

            lax.fori_loop(0, _CH, q_body, 0)
            out_copy(c, b).start()

        for b in range(nbuf):
            start_chunk(b, b)
        for c in range(nch):
            b = c % nbuf
            if c >= nbuf:
                out_copy(c - nbuf, b).wait()
            compute_chunk(c, b)
            if c + nbuf < nch:
                start_chunk(c + nbuf, b)
        for c in range(max(nch - nbuf, 0), nch):
            out_copy(c, c % nbuf).wait()

    return body


@functools.lru_cache(maxsize=2)
def _make_gather(m_slab):
    @functools.partial(
        pl.kernel,
        mesh=plsc.VectorSubcoreMesh(core_axis_name="c", subcore_axis_name="s"),
        out_type=jax.ShapeDtypeStruct((m_slab, _DIN), jnp.float32),
        scratch_types=[
            pltpu.VMEM((_NBUF, _CH * 3), jnp.int32),
            pltpu.VMEM((_NBUF, _K, _CH, 16), jnp.float32),
            pltpu.VMEM((_NBUF, _CH * 3, _DIN), jnp.float32),
            pltpu.VMEM((_NBUF, _CH, _DIN), jnp.float32),
            pltpu.SemaphoreType.DMA,
            pltpu.SemaphoreType.DMA,
            pltpu.SemaphoreType.DMA,
            pltpu.SemaphoreType.DMA,
        ],
    )
    def _gather(x_hbm, idx_hbm, w0_hbm, w1_hbm, w2_hbm, xi_hbm,
                idx_v, w_v, rows_v, out_v, g0, g1, o0, o1):
        _make_gather_body(m_slab)(x_hbm, idx_hbm, w0_hbm, w1_hbm, w2_hbm,
                                  xi_hbm, idx_v, w_v, rows_v, out_v,
                                  (g0, g1), (o0, o1))

    return _gather


_TMC = 1024


def _mlp_body(xi_ref, xs_ref, w1_ref, w2_ref, b_ref, o_ref):
    h = jnp.dot(xi_ref[...], w1_ref[...], preferred_element_type=jnp.float32)
    h = h + jnp.dot(xs_ref[...], w2_ref[...], preferred_element_type=jnp.float32)
    o_ref[...] = jnp.maximum(h + b_ref[...], 0.0)


def _mlp(xi, x_skip, W1, W2, b2d):
    m = xi.shape[0]
    return pl.pallas_call(
        _mlp_body,
        grid=(m // _TMC,),
        in_specs=[
            pl.BlockSpec((_TMC, _DIN), lambda i: (i, 0)),
            pl.BlockSpec((_TMC, _DSKIP), lambda i: (i, 0)),
            pl.BlockSpec((_DIN, _DOUT), lambda i: (0, 0)),
            pl.BlockSpec((_DSKIP, _DOUT), lambda i: (0, 0)),
            pl.BlockSpec((1, _DOUT), lambda i: (0, 0)),
        ],
        out_specs=pl.BlockSpec((_TMC, _DOUT), lambda i: (i, 0)),
        out_shape=jax.ShapeDtypeStruct((m, _DOUT), jnp.float32),
    )(xi, x_skip, W1, W2, b2d)


_NSLAB = 4


def kernel(x, pos, batch, x_skip, pos_skip, batch_skip, W, b):
    posT = pos.T
    W1, W2, b2d = W[:_DIN], W[_DIN:], b.reshape(1, _DOUT)
    ms = _M // _NSLAB

    outs = [None] * _NSLAB
    xis = [None] * _NSLAB
    nxt = _knn(pos_skip[:ms], posT)
    for s in range(_NSLAB):
        idx, w0, w1, w2 = nxt
        xis[s] = _make_gather(ms)(x, idx.reshape(-1), w0, w1, w2)
        if s + 1 < _NSLAB:
            nxt = _knn(pos_skip[(s + 1) * ms:(s + 2) * ms], posT)
        if s > 0:
            outs[s - 1] = _mlp(xis[s - 1], x_skip[(s - 1) * ms:s * ms],
                               W1, W2, b2d)
    outs[_NSLAB - 1] = _mlp(xis[_NSLAB - 1], x_skip[(_NSLAB - 1) * ms:],
                            W1, W2, b2d)
    out = jnp.concatenate(outs, axis=0)
    return (out, pos_skip, batch_skip)


# speedup vs baseline: 1.3965x; 1.0604x over previous
import functools

import jax
import jax.numpy as jnp
from jax import lax
from jax.experimental import pallas as pl
from jax.experimental.pallas import tpu as pltpu
from jax.experimental.pallas import tpu_sc as plsc

_N = 4096
_M = 16384
_DIN = 256
_DSKIP = 128
_DOUT = 256
_K = 3


_TM = 512


def _knn_body(ps_ref, posT_ref, idx_ref, w0_ref, w1_ref, w2_ref):
    ps = ps_ref[...]
    posT = posT_ref[...]
    qq = jnp.sum(ps * ps, axis=1, keepdims=True)
    kk = jnp.sum(posT * posT, axis=0, keepdims=True)
    cross = jnp.dot(ps, posT, preferred_element_type=jnp.float32)
    d2 = qq + kk - 2.0 * cross

    iota = lax.broadcasted_iota(jnp.int32, d2.shape, 1)
    big = jnp.float32(3.4e38)
    vals, idxs = [], []
    cur = d2
    for r in range(_K):
        m = jnp.min(cur, axis=1, keepdims=True)
        ival = jnp.min(jnp.where(cur <= m, iota, _N), axis=1, keepdims=True)
        vals.append(m)
        idxs.append(ival)
        if r + 1 < _K:
            cur = jnp.where(iota == ival, big, cur)

    wk = [1.0 / jnp.maximum(jnp.maximum(v, 0.0), 1e-16) for v in vals]
    wsum = wk[0] + wk[1] + wk[2]
    idx_ref[...] = jnp.concatenate(idxs, axis=1)
    w0_ref[...] = jnp.broadcast_to(wk[0] / wsum, (wk[0].shape[0], 16))
    w1_ref[...] = jnp.broadcast_to(wk[1] / wsum, (wk[1].shape[0], 16))
    w2_ref[...] = jnp.broadcast_to(wk[2] / wsum, (wk[2].shape[0], 16))


def _knn(pos_skip, posT, s, ms):
    off = s * (ms // _TM)
    wspec = pl.BlockSpec((_TM, 16), lambda i: (i, 0))
    wshape = jax.ShapeDtypeStruct((ms, 16), jnp.float32)
    return pl.pallas_call(
        _knn_body,
        grid=(ms // _TM,),
        in_specs=[
            pl.BlockSpec((_TM, 3), lambda i: (off + i, 0)),
            pl.BlockSpec((3, _N), lambda i: (0, 0)),
        ],
        out_specs=[
            pl.BlockSpec((_TM, _K), lambda i: (i, 0)),
            wspec, wspec, wspec,
        ],
        out_shape=[
            jax.ShapeDtypeStruct((ms, _K), jnp.int32),
            wshape, wshape, wshape,
        ],
    )(pos_skip, posT)


_NW = 32
_CH = 32
_NBUF = 2


def _make_gather_body(m_slab):
    qw = m_slab // _NW
    nch = qw // _CH
    nbuf = min(_NBUF, nch)

    def body(x_hbm, idx_hbm, w0_hbm, w1_hbm, w2_hbm, xi_hbm,
             idx_v, w_v, rows_v, out_v, gsems, osems):
        wid = lax.axis_index("s") * 2 + lax.axis_index("c")
        q0 = wid * qw

        def out_copy(c, b):
            return pltpu.make_async_copy(
                out_v.at[b], xi_hbm.at[pl.ds(q0 + c * _CH, _CH)], osems[b])

        def start_chunk(c, b):
            qb = q0 + c * _CH
            pltpu.sync_copy(idx_hbm.at[pl.ds(qb * 3, _CH * 3)], idx_v.at[b])
            pltpu.sync_copy(w0_hbm.at[pl.ds(qb, _CH)], w_v.at[b, 0])
            pltpu.sync_copy(w1_hbm.at[pl.ds(qb, _CH)], w_v.at[b, 1])
            pltpu.sync_copy(w2_hbm.at[pl.ds(qb, _CH)], w_v.at[b, 2])
            pltpu.make_async_copy(x_hbm.at[idx_v.at[b]], rows_v.at[b],
                                  gsems[b]).start()

        def compute_chunk(c, b):
            pltpu.make_async_copy(x_hbm.at[idx_v.at[b]], rows_v.at[b],
                                  gsems[b]).wait()

            def q_body(q, carry):
                w0 = w_v[b, 0, q]
                w1 = w_v[b, 1, q]
                w2 = w_v[b, 2, q]
                for d in range(_DIN // 16):
                    sl = pl.ds(16 * d, 16)
                    out_v[b, q, sl] = (w0 * rows_v[b, 3 * q, sl]
                                       + w1 * rows_v[b, 3 * q + 1, sl]
                                       + w2 * rows_v[b, 3 * q + 2, sl])
                return carry

            lax.fori_loop(0, _CH, q_body, 0)
            out_copy(c, b).start()

        for b in range(nbuf):
            start_chunk(b, b)
        for c in range(nch):
            b = c % nbuf
            if c >= nbuf:
                out_copy(c - nbuf, b).wait()
            compute_chunk(c, b)
            if c + nbuf < nch:
                start_chunk(c + nbuf, b)
        for c in range(max(nch - nbuf, 0), nch):
            out_copy(c, c % nbuf).wait()

    return body


@functools.lru_cache(maxsize=2)
def _make_gather(m_slab):
    @functools.partial(
        pl.kernel,
        mesh=plsc.VectorSubcoreMesh(core_axis_name="c", subcore_axis_name="s"),
        out_type=jax.ShapeDtypeStruct((m_slab, _DIN), jnp.float32),
        scratch_types=[
            pltpu.VMEM((_NBUF, _CH * 3), jnp.int32),
            pltpu.VMEM((_NBUF, _K, _CH, 16), jnp.float32),
            pltpu.VMEM((_NBUF, _CH * 3, _DIN), jnp.float32),
            pltpu.VMEM((_NBUF, _CH, _DIN), jnp.float32),
            pltpu.SemaphoreType.DMA,
            pltpu.SemaphoreType.DMA,
            pltpu.SemaphoreType.DMA,
            pltpu.SemaphoreType.DMA,
        ],
    )
    def _gather(x_hbm, idx_hbm, w0_hbm, w1_hbm, w2_hbm, xi_hbm,
                idx_v, w_v, rows_v, out_v, g0, g1, o0, o1):
        _make_gather_body(m_slab)(x_hbm, idx_hbm, w0_hbm, w1_hbm, w2_hbm,
                                  xi_hbm, idx_v, w_v, rows_v, out_v,
                                  (g0, g1), (o0, o1))

    return _gather


_TMC = 1024


def _mlp_body(xi_ref, xs_ref, w1_ref, w2_ref, b_ref, o_ref):
    h = jnp.dot(xi_ref[...], w1_ref[...], preferred_element_type=jnp.float32)
    h = h + jnp.dot(xs_ref[...], w2_ref[...], preferred_element_type=jnp.float32)
    o_ref[...] = jnp.maximum(h + b_ref[...], 0.0)


def _mlp_into(buf, xi, x_skip, W1, W2, b2d, s, ms):
    off = s * (ms // _TMC)

    def body(*refs):
        _mlp_body(*refs[-6:])

    specs = [
        pl.BlockSpec((_TMC, _DIN), lambda i: (i, 0)),
        pl.BlockSpec((_TMC, _DSKIP), lambda i: (off + i, 0)),
        pl.BlockSpec((_DIN, _DOUT), lambda i: (0, 0)),
        pl.BlockSpec((_DSKIP, _DOUT), lambda i: (0, 0)),
        pl.BlockSpec((1, _DOUT), lambda i: (0, 0)),
    ]
    args = (xi, x_skip, W1, W2, b2d)
    aliases = {}
    if buf is not None:
        specs = [pl.BlockSpec((_TMC, _DOUT), lambda i: (off + i, 0))] + specs
        args = (buf,) + args
        aliases = {0: 0}
    return pl.pallas_call(
        body,
        grid=(ms // _TMC,),
        in_specs=specs,
        out_specs=pl.BlockSpec((_TMC, _DOUT), lambda i: (off + i, 0)),
        out_shape=jax.ShapeDtypeStruct((_M, _DOUT), jnp.float32),
        input_output_aliases=aliases,
    )(*args)


_NSLAB = 4


def kernel(x, pos, batch, x_skip, pos_skip, batch_skip, W, b):
    posT = pos.T
    W1, W2, b2d = W[:_DIN], W[_DIN:], b.reshape(1, _DOUT)
    ms = _M // _NSLAB

    xis = [None] * _NSLAB
    out = None
    nxt = _knn(pos_skip, posT, 0, ms)
    for s in range(_NSLAB):
        idx, w0, w1, w2 = nxt
        xis[s] = _make_gather(ms)(x, idx.reshape(-1), w0, w1, w2)
        if s + 1 < _NSLAB:
            nxt = _knn(pos_skip, posT, s + 1, ms)
        if s > 0:
            out = _mlp_into(out, xis[s - 1], x_skip, W1, W2, b2d, s - 1, ms)
    out = _mlp_into(out, xis[_NSLAB - 1], x_skip, W1, W2, b2d, _NSLAB - 1, ms)
    return (out, pos_skip, batch_skip)
